# baseline (speedup 1.0000x reference)
import functools

import jax
import jax.numpy as jnp
from jax import lax
from jax.experimental import pallas as pl
from jax.experimental.pallas import tpu as pltpu

BATCH = 16384
ATOMS = 200
NUM_SPECIES = 8

BR = 512


def _tc_body(tab_ref, spec_ref, en_ref, out_ref):
    x = spec_ref[...]
    acc = jnp.zeros(x.shape, jnp.float32)
    for k in range(NUM_SPECIES):
        acc = jnp.where(x == k, tab_ref[k], acc)
    ones = jnp.ones((ATOMS,), jnp.float32)
    rowsum = jax.lax.dot_general(
        acc, ones, (((1,), (0,)), ((), ())),
        preferred_element_type=jnp.float32)
    out_ref[...] = en_ref[...] + rowsum


@functools.partial(jax.jit)
def _tc_shift(species, energies, self_energies):
    grid = (BATCH // BR,)
    return pl.pallas_call(
        _tc_body,
        grid=grid,
        in_specs=[
            pl.BlockSpec(memory_space=pltpu.SMEM),
            pl.BlockSpec((BR, ATOMS), lambda i: (i, 0)),
            pl.BlockSpec((BR,), lambda i: (i,)),
        ],
        out_specs=pl.BlockSpec((BR,), lambda i: (i,)),
        out_shape=jax.ShapeDtypeStruct((BATCH,), jnp.float32),
        compiler_params=pltpu.CompilerParams(
            dimension_semantics=("arbitrary",)),
    )(self_energies, species, energies)


def kernel(species, energies, self_energies):
    shifted = _tc_shift(species, energies, self_energies)
    return (species, shifted)


# speedup vs baseline: 1.2165x; 1.2165x over previous
import functools

import jax
import jax.numpy as jnp
from jax import lax
from jax.experimental import pallas as pl
from jax.experimental.pallas import tpu as pltpu

BATCH = 16384
ATOMS = 200
NUM_SPECIES = 8

BR = 2048


def _tc_body(tab_ref, spec_ref, en_ref, out_ref):
    x = spec_ref[...]
    acc = jnp.zeros(x.shape, jnp.float32)
    for k in range(NUM_SPECIES):
        acc = jnp.where(x == k, tab_ref[k], acc)
    ones = jnp.ones((ATOMS,), jnp.float32)
    rowsum = jax.lax.dot_general(
        acc, ones, (((1,), (0,)), ((), ())),
        preferred_element_type=jnp.float32)
    out_ref[...] = en_ref[...] + rowsum


@functools.partial(jax.jit)
def _tc_shift(species, energies, self_energies):
    grid = (BATCH // BR,)
    return pl.pallas_call(
        _tc_body,
        grid=grid,
        in_specs=[
            pl.BlockSpec(memory_space=pltpu.SMEM),
            pl.BlockSpec((BR, ATOMS), lambda i: (i, 0)),
            pl.BlockSpec((BR,), lambda i: (i,)),
        ],
        out_specs=pl.BlockSpec((BR,), lambda i: (i,)),
        out_shape=jax.ShapeDtypeStruct((BATCH,), jnp.float32),
        compiler_params=pltpu.CompilerParams(
            dimension_semantics=("arbitrary",)),
    )(self_energies, species, energies)


def kernel(species, energies, self_energies):
    shifted = _tc_shift(species, energies, self_energies)
    return (species, shifted)
